# baseline (speedup 1.0000x reference)
import functools

import jax
import jax.numpy as jnp
from jax import lax
from jax.experimental import pallas as pl
from jax.experimental.pallas import tpu as pltpu
from jax.experimental.pallas import tpu_sc as plsc


@functools.lru_cache(maxsize=None)
def _build(batch: int, n_rows: int, dim: int):
    info = plsc.get_sparse_core_info()
    nc, ns = info.num_cores, info.num_subcores
    nw = nc * ns
    assert batch % (8 * nw) == 0, (batch, nw)
    b_per_w = batch // nw
    mesh = plsc.VectorSubcoreMesh(core_axis_name="c", subcore_axis_name="s")

    nck = 2
    cb = b_per_w // nck

    @functools.partial(
        pl.kernel,
        mesh=mesh,
        out_type=jax.ShapeDtypeStruct((batch, dim), jnp.float32),
        compiler_params=pltpu.CompilerParams(
            disable_bounds_checks=True,
            disable_semaphore_checks=True,
        ),
        scratch_types=[pltpu.VMEM((cb,), jnp.int32) for _ in range(nck)]
        + [pltpu.VMEM((cb, dim), jnp.float32) for _ in range(nck)]
        + [pltpu.SemaphoreType.DMA for _ in range(3 * nck)],
    )
    def k(idx_hbm, table_hbm, out_hbm, *rest):
        idxs = rest[:nck]
        bufs = rest[nck : 2 * nck]
        isems = rest[2 * nck : 3 * nck]
        gsems = rest[3 * nck : 4 * nck]
        osems = rest[4 * nck :]
        wid = lax.axis_index("s") * nc + lax.axis_index("c")
        base = wid * b_per_w
        icps = [
            pltpu.async_copy(
                idx_hbm.at[pl.ds(base + j * cb, cb)], idxs[j], isems[j]
            )
            for j in range(nck)
        ]
        gcps = []
        for j in range(nck):
            icps[j].wait()
            gcps.append(
                pltpu.async_copy(table_hbm.at[idxs[j]], bufs[j], gsems[j])
            )
        ocps = []
        for j in range(nck):
            gcps[j].wait()
            ocps.append(
                pltpu.async_copy(
                    bufs[j], out_hbm.at[pl.ds(base + j * cb, cb)], osems[j]
                )
            )
        for o in ocps:
            o.wait()

    return k


def kernel(gene_ids, embedding_weight):
    (batch,) = gene_ids.shape
    n_rows, dim = embedding_weight.shape
    k = _build(batch, n_rows, dim)
    return k(gene_ids.astype(jnp.int32), embedding_weight)


# speedup vs baseline: 1.0031x; 1.0031x over previous
import functools

import jax
import jax.numpy as jnp
from jax import lax
from jax.experimental import pallas as pl
from jax.experimental.pallas import tpu as pltpu
from jax.experimental.pallas import tpu_sc as plsc


@functools.lru_cache(maxsize=None)
def _build(batch: int, n_rows: int, dim: int):
    info = plsc.get_sparse_core_info()
    nc, ns = info.num_cores, info.num_subcores
    nw = nc * ns
    assert batch % (8 * nw) == 0, (batch, nw)
    b_per_w = batch // nw
    mesh = plsc.VectorSubcoreMesh(core_axis_name="c", subcore_axis_name="s")

    @functools.partial(
        pl.kernel,
        mesh=mesh,
        out_type=jax.ShapeDtypeStruct((batch, dim), jnp.float32),
        scratch_types=[
            pltpu.VMEM((b_per_w,), jnp.int32),
            pltpu.VMEM((b_per_w, dim), jnp.float32),
            pltpu.SemaphoreType.DMA,
        ],
    )
    def k(idx_hbm, table_hbm, out_hbm, idx_v, rows_v, sem):
        wid = lax.axis_index("s") * nc + lax.axis_index("c")
        base = wid * b_per_w
        pltpu.sync_copy(idx_hbm.at[pl.ds(base, b_per_w)], idx_v)
        pltpu.async_copy(table_hbm.at[idx_v], rows_v, sem).wait()
        pltpu.sync_copy(rows_v, out_hbm.at[pl.ds(base, b_per_w)])

    return k


def kernel(gene_ids, embedding_weight):
    (batch,) = gene_ids.shape
    n_rows, dim = embedding_weight.shape
    k = _build(batch, n_rows, dim)
    return k(gene_ids.astype(jnp.int32), embedding_weight)
